# baseline (speedup 1.0000x reference)
import jax
import jax.numpy as jnp
from jax import lax
from jax.experimental import pallas as pl
from jax.experimental.pallas import tpu as pltpu
from jax.experimental.pallas import tpu_sc as plsc

B = 4
L = 8192
D = 128
NC = 2
NS = 16
NW = NC * NS
LPW = L // NW
CHUNK = 32
GROWS = B * CHUNK
NG = LPW // CHUNK
NBG = 5
LANES = D // 16


def _body(ids_hbm, embed_hbm, type_hbm, pos_hbm, out_hbm,
          idx_raw, gidx, type_v, pos_v, gbuf,
          sem_idx, sem_pt, sg0, sg1, sg2, sg3, sg4,
          ss0, ss1, ss2, ss3, ss4):
    wid = lax.axis_index("s") * NC + lax.axis_index("c")
    l_base = wid * LPW

    idx_h = [pltpu.async_copy(ids_hbm.at[b, pl.ds(l_base, LPW)],
                              idx_raw.at[b], sem_idx)
             for b in range(B)]
    pos_h = pltpu.async_copy(pos_hbm.at[pl.ds(l_base, LPW)], pos_v, sem_pt)
    typ_h = pltpu.async_copy(type_hbm.at[0], type_v, sem_pt)
    for h in idx_h:
        h.wait()

    for g in range(NG):
        for b in range(B):
            for j in range(CHUNK // 16):
                gidx[g, pl.ds(b * CHUNK + j * 16, 16)] = (
                    idx_raw[b, pl.ds(g * CHUNK + j * 16, 16)])

    sg = (sg0, sg1, sg2, sg3, sg4)
    ss = (ss0, ss1, ss2, ss3, ss4)

    def issue_gather(g):
        par = g % NBG
        return pltpu.async_copy(embed_hbm.at[gidx.at[g]], gbuf.at[par],
                                sg[par])

    PF = NBG - 1
    g_handles = [None] * NBG
    s_handles = [None] * NBG
    for g in range(PF):
        g_handles[g] = issue_gather(g)

    pos_h.wait()
    typ_h.wait()
    tvecs = [type_v[pl.ds(j * 16, 16)] for j in range(LANES)]

    for g in range(NG):
        par = g % NBG
        g_handles[par].wait()

        boff = g * CHUNK

        def row_body(i, carry):
            for u in range(2):
                r = i * 2 + u
                bias = [pos_v[boff + r, pl.ds(j * 16, 16)] + tvecs[j]
                        for j in range(LANES)]
                for b in range(B):
                    for j in range(LANES):
                        sl = pl.ds(j * 16, 16)
                        gbuf[par, b * CHUNK + r, sl] = (
                            gbuf[par, b * CHUNK + r, sl] + bias[j])
            return carry

        lax.fori_loop(0, CHUNK // 2, row_body, 0)

        s_handles[par] = pltpu.async_copy(
            gbuf.at[par].reshape(B, CHUNK, D),
            out_hbm.at[pl.ds(0, B), pl.ds(l_base + g * CHUNK, CHUNK)],
            ss[par])

        if g + PF < NG:
            npar = (g + PF) % NBG
            if s_handles[npar] is not None:
                s_handles[npar].wait()
                s_handles[npar] = None
            g_handles[npar] = issue_gather(g + PF)

    for h in s_handles:
        if h is not None:
            h.wait()


_emb_lookup = pl.kernel(
    _body,
    out_type=jax.ShapeDtypeStruct((B, L, D), jnp.float32),
    mesh=plsc.VectorSubcoreMesh(core_axis_name="c", subcore_axis_name="s",
                                num_cores=NC, num_subcores=NS),
    scratch_types=[
        pltpu.VMEM((B, LPW), jnp.int32),
        pltpu.VMEM((NG, GROWS), jnp.int32),
        pltpu.VMEM((D,), jnp.float32),
        pltpu.VMEM((LPW, D), jnp.float32),
        pltpu.VMEM((NBG, GROWS, D), jnp.float32),
        pltpu.SemaphoreType.DMA,
        pltpu.SemaphoreType.DMA,
        pltpu.SemaphoreType.DMA,
        pltpu.SemaphoreType.DMA,
        pltpu.SemaphoreType.DMA,
        pltpu.SemaphoreType.DMA,
        pltpu.SemaphoreType.DMA,
        pltpu.SemaphoreType.DMA,
        pltpu.SemaphoreType.DMA,
        pltpu.SemaphoreType.DMA,
        pltpu.SemaphoreType.DMA,
        pltpu.SemaphoreType.DMA,
    ],
)


@jax.jit
def kernel(input_ids, embed_weight, type_weight, pos_weight):
    return _emb_lookup(input_ids.astype(jnp.int32),
                       embed_weight, type_weight, pos_weight)


# speedup vs baseline: 1.0467x; 1.0467x over previous
import jax
import jax.numpy as jnp
from jax import lax
from jax.experimental import pallas as pl
from jax.experimental.pallas import tpu as pltpu
from jax.experimental.pallas import tpu_sc as plsc

B = 4
L = 8192
D = 128
NC = 2
NS = 16
NW = NC * NS
LPW = L // NW
CHUNK = 32
GROWS = B * CHUNK
NG = LPW // CHUNK
NBG = 5
LANES = D // 16


def _body(ids_hbm, embed_hbm, type_hbm, pos_hbm, out_hbm,
          idx_raw, gidx, type_v, pos_v, gbuf,
          sem_idx, sem_pt, sg0, sg1, sg2, sg3, sg4,
          ss0, ss1, ss2, ss3, ss4):
    wid = lax.axis_index("s") * NC + lax.axis_index("c")
    l_base = wid * LPW

    idx_h = pltpu.async_copy(ids_hbm.at[pl.ds(0, B), pl.ds(l_base, LPW)],
                             idx_raw, sem_idx)
    pos_h = pltpu.async_copy(pos_hbm.at[pl.ds(l_base, LPW)], pos_v, sem_pt)
    typ_h = pltpu.async_copy(type_hbm.at[0], type_v, sem_pt)
    idx_h.wait()

    for g in range(NG):
        for b in range(B):
            for j in range(CHUNK // 16):
                gidx[g, pl.ds(b * CHUNK + j * 16, 16)] = (
                    idx_raw[b, pl.ds(g * CHUNK + j * 16, 16)])

    sg = (sg0, sg1, sg2, sg3, sg4)
    ss = (ss0, ss1, ss2, ss3, ss4)

    def issue_gather(g):
        par = g % NBG
        return pltpu.async_copy(embed_hbm.at[gidx.at[g]], gbuf.at[par],
                                sg[par])

    PF = NBG - 1
    g_handles = [None] * NBG
    s_handles = [None] * NBG
    for g in range(PF):
        g_handles[g] = issue_gather(g)

    pos_h.wait()
    typ_h.wait()
    tvecs = [type_v[pl.ds(j * 16, 16)] for j in range(LANES)]

    for g in range(NG):
        par = g % NBG
        g_handles[par].wait()

        boff = g * CHUNK

        def row_body(r, carry):
            bias = [pos_v[boff + r, pl.ds(j * 16, 16)] + tvecs[j]
                    for j in range(LANES)]
            for b in range(B):
                for j in range(LANES):
                    sl = pl.ds(j * 16, 16)
                    gbuf[par, b * CHUNK + r, sl] = (
                        gbuf[par, b * CHUNK + r, sl] + bias[j])
            return carry

        lax.fori_loop(0, CHUNK, row_body, 0)

        s_handles[par] = pltpu.async_copy(
            gbuf.at[par].reshape(B, CHUNK, D),
            out_hbm.at[pl.ds(0, B), pl.ds(l_base + g * CHUNK, CHUNK)],
            ss[par])

        if g + PF < NG:
            npar = (g + PF) % NBG
            if s_handles[npar] is not None:
                s_handles[npar].wait()
                s_handles[npar] = None
            g_handles[npar] = issue_gather(g + PF)

    for h in s_handles:
        if h is not None:
            h.wait()


_emb_lookup = pl.kernel(
    _body,
    out_type=jax.ShapeDtypeStruct((B, L, D), jnp.float32),
    mesh=plsc.VectorSubcoreMesh(core_axis_name="c", subcore_axis_name="s",
                                num_cores=NC, num_subcores=NS),
    scratch_types=[
        pltpu.VMEM((B, LPW), jnp.int32),
        pltpu.VMEM((NG, GROWS), jnp.int32),
        pltpu.VMEM((D,), jnp.float32),
        pltpu.VMEM((LPW, D), jnp.float32),
        pltpu.VMEM((NBG, GROWS, D), jnp.float32),
        pltpu.SemaphoreType.DMA,
        pltpu.SemaphoreType.DMA,
        pltpu.SemaphoreType.DMA,
        pltpu.SemaphoreType.DMA,
        pltpu.SemaphoreType.DMA,
        pltpu.SemaphoreType.DMA,
        pltpu.SemaphoreType.DMA,
        pltpu.SemaphoreType.DMA,
        pltpu.SemaphoreType.DMA,
        pltpu.SemaphoreType.DMA,
        pltpu.SemaphoreType.DMA,
        pltpu.SemaphoreType.DMA,
    ],
)


@jax.jit
def kernel(input_ids, embed_weight, type_weight, pos_weight):
    return _emb_lookup(input_ids.astype(jnp.int32),
                       embed_weight, type_weight, pos_weight)


# speedup vs baseline: 1.0515x; 1.0046x over previous
import jax
import jax.numpy as jnp
from jax import lax
from jax.experimental import pallas as pl
from jax.experimental.pallas import tpu as pltpu
from jax.experimental.pallas import tpu_sc as plsc

B = 4
L = 8192
D = 128
NC = 2
NS = 16
NW = NC * NS
LPW = L // NW
CHUNK = 32
GROWS = B * CHUNK
NG = LPW // CHUNK
NBG = 5
LANES = D // 16


def _body(ids_hbm, embed_hbm, type_hbm, pos_hbm, out_hbm,
          idx_raw, gidx, type_v, pos_v, gbuf,
          sem_idx, sem_pt, sg0, sg1, sg2, sg3, sg4,
          ss0, ss1, ss2, ss3, ss4):
    wid = lax.axis_index("s") * NC + lax.axis_index("c")
    l_base = wid * LPW

    idx_h = pltpu.async_copy(ids_hbm.at[pl.ds(0, B), pl.ds(l_base, LPW)],
                             idx_raw, sem_idx)
    pos_h = pltpu.async_copy(pos_hbm.at[pl.ds(l_base, LPW)], pos_v, sem_pt)
    typ_h = pltpu.async_copy(type_hbm.at[0], type_v, sem_pt)
    idx_h.wait()

    for g in range(NG):
        for b in range(B):
            for j in range(CHUNK // 16):
                gidx[g, pl.ds(b * CHUNK + j * 16, 16)] = (
                    idx_raw[b, pl.ds(g * CHUNK + j * 16, 16)])

    sg = (sg0, sg1, sg2, sg3, sg4)
    ss = (ss0, ss1, ss2, ss3, ss4)

    def issue_gather(g):
        par = g % NBG
        return pltpu.async_copy(embed_hbm.at[gidx.at[g]], gbuf.at[par],
                                sg[par])

    PF = NBG - 1
    g_handles = [None] * NBG
    s_handles = [None] * NBG
    for g in range(PF):
        g_handles[g] = issue_gather(g)

    pos_h.wait()
    typ_h.wait()
    tvecs = [type_v[pl.ds(j * 16, 16)] for j in range(LANES)]

    for g in range(NG):
        par = g % NBG
        g_handles[par].wait()

        boff = g * CHUNK

        def row_body(r, carry):
            bias = [pos_v[boff + r, pl.ds(j * 16, 16)] + tvecs[j]
                    for j in range(LANES)]
            for b in range(B):
                for j in range(LANES):
                    sl = pl.ds(j * 16, 16)
                    plsc.addupdate(gbuf.at[par, b * CHUNK + r, sl], bias[j])
            return carry

        lax.fori_loop(0, CHUNK, row_body, 0)

        s_handles[par] = pltpu.async_copy(
            gbuf.at[par].reshape(B, CHUNK, D),
            out_hbm.at[pl.ds(0, B), pl.ds(l_base + g * CHUNK, CHUNK)],
            ss[par])

        if g + PF < NG:
            npar = (g + PF) % NBG
            if s_handles[npar] is not None:
                s_handles[npar].wait()
                s_handles[npar] = None
            g_handles[npar] = issue_gather(g + PF)

    for h in s_handles:
        if h is not None:
            h.wait()


_emb_lookup = pl.kernel(
    _body,
    out_type=jax.ShapeDtypeStruct((B, L, D), jnp.float32),
    mesh=plsc.VectorSubcoreMesh(core_axis_name="c", subcore_axis_name="s",
                                num_cores=NC, num_subcores=NS),
    scratch_types=[
        pltpu.VMEM((B, LPW), jnp.int32),
        pltpu.VMEM((NG, GROWS), jnp.int32),
        pltpu.VMEM((D,), jnp.float32),
        pltpu.VMEM((LPW, D), jnp.float32),
        pltpu.VMEM((NBG, GROWS, D), jnp.float32),
        pltpu.SemaphoreType.DMA,
        pltpu.SemaphoreType.DMA,
        pltpu.SemaphoreType.DMA,
        pltpu.SemaphoreType.DMA,
        pltpu.SemaphoreType.DMA,
        pltpu.SemaphoreType.DMA,
        pltpu.SemaphoreType.DMA,
        pltpu.SemaphoreType.DMA,
        pltpu.SemaphoreType.DMA,
        pltpu.SemaphoreType.DMA,
        pltpu.SemaphoreType.DMA,
        pltpu.SemaphoreType.DMA,
    ],
)


@jax.jit
def kernel(input_ids, embed_weight, type_weight, pos_weight):
    return _emb_lookup(input_ids.astype(jnp.int32),
                       embed_weight, type_weight, pos_weight)
